# baseline (speedup 1.0000x reference)
import functools

import jax
import jax.numpy as jnp
from jax import lax
from jax.experimental import pallas as pl
from jax.experimental.pallas import tpu as pltpu
from jax.experimental.pallas import tpu_sc as plsc

NS = 16
NC = 2
K = 128


def _mm(xs, W, b, *, relu, split):
    N = xs[0].shape[0]
    Dout = W.shape[1]
    BR = max(br for br in (2048, 2000, 1024, 1000, 512, 400, 80, 8, 1)
             if N % br == 0)
    grid = (N // BR,)
    b2 = b.reshape(1, Dout)
    in_specs = [pl.BlockSpec((BR, x.shape[1]), lambda i: (i, 0)) for x in xs]
    in_specs.append(pl.BlockSpec(W.shape, lambda i: (0, 0)))
    in_specs.append(pl.BlockSpec((1, Dout), lambda i: (0, 0)))
    if split:
        H = Dout // 2
        out_shape = (jax.ShapeDtypeStruct((N, H), jnp.float32),
                     jax.ShapeDtypeStruct((N, H), jnp.float32))
        out_specs = (pl.BlockSpec((BR, H), lambda i: (i, 0)),
                     pl.BlockSpec((BR, H), lambda i: (i, 0)))
    else:
        out_shape = jax.ShapeDtypeStruct((N, Dout), jnp.float32)
        out_specs = pl.BlockSpec((BR, Dout), lambda i: (i, 0))
    nx = len(xs)

    def body(*refs):
        if nx == 1:
            x = refs[0][...]
        else:
            x = jnp.concatenate([refs[0][...], refs[1][...]], axis=1)
        w = refs[nx][...]
        bias = refs[nx + 1][...]
        r = jnp.dot(x, w, preferred_element_type=jnp.float32) + bias
        if relu:
            r = jnp.maximum(r, 0.0)
        if split:
            H = r.shape[1] // 2
            refs[nx + 2][...] = r[:, :H]
            refs[nx + 3][...] = r[:, H:]
        else:
            refs[nx + 2][...] = r

    return pl.pallas_call(body, grid=grid, in_specs=in_specs,
                          out_specs=out_specs, out_shape=out_shape)(*xs, W, b2)


@functools.lru_cache(maxsize=None)
def _make_sc_agg(N, Lh, CH):
    R0 = (N // (8 * NS)) * 8
    rem = N - R0 * NS
    mesh = plsc.VectorSubcoreMesh(core_axis_name="c", subcore_axis_name="s",
                                  num_cores=NC, num_subcores=NS)

    @functools.partial(
        pl.kernel,
        mesh=mesh,
        out_type=(jax.ShapeDtypeStruct((N, Lh), jnp.float32),
                  jax.ShapeDtypeStruct((N, Lh), jnp.float32)),
        scratch_types=[
            pltpu.VMEM((2, K), jnp.int32),
            pltpu.VMEM((2, K), jnp.int32),
            pltpu.VMEM((K, Lh), jnp.float32),
            pltpu.VMEM((K, Lh), jnp.float32),
            pltpu.VMEM_SHARED((N + 8, Lh), jnp.float32),
            pltpu.SemaphoreType.DMA,
            pltpu.SemaphoreType.DMA,
            pltpu.SemaphoreType.DMA,
            pltpu.SemaphoreType.DMA,
        ],
    )
    def sc_agg(h0, h1, src_hbm, dst_hbm, o0, o1,
               src_v, dst_v, rows_v0, rows_v1, agg_sh,
               semg0, semg1, semi0, semi1):
        c = lax.axis_index("c")
        s = lax.axis_index("s")

        def run(h_hbm, o_hbm):
            base = s * R0
            pltpu.sync_copy(h_hbm.at[pl.ds(base, R0)],
                            agg_sh.at[pl.ds(base, R0)])
            if rem:
                @pl.when(s == NS - 1)
                def _():
                    pltpu.sync_copy(h_hbm.at[pl.ds(R0 * NS, rem)],
                                    agg_sh.at[pl.ds(R0 * NS, rem)])
            plsc.subcore_barrier()

            src_s = src_hbm.at[s]
            dst_s = dst_hbm.at[s]

            def step(j, sv, dv, rv, semg, semi,
                     sv_n, dv_n, rv_n, semg_n, semi_n):
                @pl.when(j + 1 < CH)
                def _():
                    pltpu.make_async_copy(src_s.at[j + 1], sv_n,
                                          semi_n).wait()
                    pltpu.make_async_copy(dst_s.at[j + 1], dv_n,
                                          semi_n).wait()
                    pltpu.async_copy(h_hbm.at[sv_n], rv_n, semg_n)

                pltpu.make_async_copy(h_hbm.at[sv], rv, semg).wait()

                @pl.when(j + 2 < CH)
                def _():
                    pltpu.async_copy(src_s.at[j + 2], sv, semi)

                pltpu.sync_copy(rv, agg_sh.at[dv], add=True)

                @pl.when(j + 2 < CH)
                def _():
                    pltpu.async_copy(dst_s.at[j + 2], dv, semi)

            slot0 = (src_v.at[0], dst_v.at[0], rows_v0, semg0, semi0)
            slot1 = (src_v.at[1], dst_v.at[1], rows_v1, semg1, semi1)

            pltpu.sync_copy(src_s.at[0], src_v.at[0])
            pltpu.sync_copy(dst_s.at[0], dst_v.at[0])
            pltpu.async_copy(h_hbm.at[src_v.at[0]], rows_v0, semg0)
            if CH > 1:
                pltpu.async_copy(src_s.at[1], src_v.at[1], semi1)
                pltpu.async_copy(dst_s.at[1], dst_v.at[1], semi1)

            def pair(jj, carry):
                j0 = jj * 2
                step(j0, *slot0, *slot1)

                @pl.when(j0 + 1 < CH)
                def _():
                    step(j0 + 1, *slot1, *slot0)

                return carry

            lax.fori_loop(0, (CH + 1) // 2, pair, 0)
            plsc.subcore_barrier()
            pltpu.sync_copy(agg_sh.at[pl.ds(base, R0)],
                            o_hbm.at[pl.ds(base, R0)])
            if rem:
                @pl.when(s == NS - 1)
                def _():
                    pltpu.sync_copy(agg_sh.at[pl.ds(R0 * NS, rem)],
                                    o_hbm.at[pl.ds(R0 * NS, rem)])

        @pl.when(c == 0)
        def _():
            run(h0, o0)

        @pl.when(c == 1)
        def _():
            run(h1, o1)

    return sc_agg


def kernel(features, edge_list, W_enc, b_enc, W_comb, b_comb, W_out, b_out):
    N = features.shape[0]
    Dlat = W_enc.shape[1]
    Lh = Dlat // 2
    E = edge_list.shape[0]
    layers = W_comb.shape[0]
    CH = -(-E // (NS * K))
    Epad = NS * K * CH

    dst = edge_list[:, 0].astype(jnp.int32)
    src = edge_list[:, 1].astype(jnp.int32)
    pad = Epad - E
    src_p = jnp.concatenate(
        [src, jnp.zeros((pad,), jnp.int32)]).reshape(NS, CH, K)
    dst_p = jnp.concatenate(
        [dst, jnp.full((pad,), N, jnp.int32)]).reshape(NS, CH, K)

    sc_agg = _make_sc_agg(N, Lh, CH)

    h0, h1 = _mm((features,), W_enc, b_enc, relu=False, split=True)
    for l in range(layers):
        s0, s1 = sc_agg(h0, h1, src_p, dst_p)
        h0, h1 = _mm((s0, s1), W_comb[l], b_comb[l], relu=True, split=True)
    return _mm((h0, h1), W_out, b_out, relu=False, split=False)


# speedup vs baseline: 1.3956x; 1.3956x over previous
import functools

import jax
import jax.numpy as jnp
from jax import lax
from jax.experimental import pallas as pl
from jax.experimental.pallas import tpu as pltpu
from jax.experimental.pallas import tpu_sc as plsc

NS = 16
NC = 2
K = 64
NB = 4
NI = 8


def _mm(xs, W, b, *, relu, split):
    N = xs[0].shape[0]
    Dout = W.shape[1]
    BR = max(br for br in (2048, 2000, 1024, 1000, 512, 400, 80, 8, 1)
             if N % br == 0)
    grid = (N // BR,)
    b2 = b.reshape(1, Dout)
    in_specs = [pl.BlockSpec((BR, x.shape[1]), lambda i: (i, 0)) for x in xs]
    in_specs.append(pl.BlockSpec(W.shape, lambda i: (0, 0)))
    in_specs.append(pl.BlockSpec((1, Dout), lambda i: (0, 0)))
    if split:
        H = Dout // 2
        out_shape = (jax.ShapeDtypeStruct((N, H), jnp.float32),
                     jax.ShapeDtypeStruct((N, H), jnp.float32))
        out_specs = (pl.BlockSpec((BR, H), lambda i: (i, 0)),
                     pl.BlockSpec((BR, H), lambda i: (i, 0)))
    else:
        out_shape = jax.ShapeDtypeStruct((N, Dout), jnp.float32)
        out_specs = pl.BlockSpec((BR, Dout), lambda i: (i, 0))
    nx = len(xs)

    def body(*refs):
        if nx == 1:
            x = refs[0][...]
        else:
            x = jnp.concatenate([refs[0][...], refs[1][...]], axis=1)
        w = refs[nx][...]
        bias = refs[nx + 1][...]
        r = jnp.dot(x, w, preferred_element_type=jnp.float32) + bias
        if relu:
            r = jnp.maximum(r, 0.0)
        if split:
            H = r.shape[1] // 2
            refs[nx + 2][...] = r[:, :H]
            refs[nx + 3][...] = r[:, H:]
        else:
            refs[nx + 2][...] = r

    return pl.pallas_call(body, grid=grid, in_specs=in_specs,
                          out_specs=out_specs, out_shape=out_shape)(*xs, W, b2)


@functools.lru_cache(maxsize=None)
def _make_sc_agg(N, Lh, CH):
    R0 = (N // (8 * NS)) * 8
    rem = N - R0 * NS
    mesh = plsc.VectorSubcoreMesh(core_axis_name="c", subcore_axis_name="s",
                                  num_cores=NC, num_subcores=NS)

    @functools.partial(
        pl.kernel,
        mesh=mesh,
        out_type=(jax.ShapeDtypeStruct((N, Lh), jnp.float32),
                  jax.ShapeDtypeStruct((N, Lh), jnp.float32)),
        scratch_types=(
            [pltpu.VMEM((NI, K), jnp.int32),
             pltpu.VMEM((NI, K), jnp.int32)]
            + [pltpu.VMEM((K, Lh), jnp.float32) for _ in range(NB)]
            + [pltpu.VMEM_SHARED((N + 8, Lh), jnp.float32)]
            + [pltpu.SemaphoreType.DMA for _ in range(2 * NB + NI)]
        ),
    )
    def sc_agg(h0, h1, src_hbm, dst_hbm, o0, o1,
               src_v, dst_v, *rest):
        rows = list(rest[:NB])
        agg_sh = rest[NB]
        semg = list(rest[NB + 1:NB + 1 + NB])
        semsc = list(rest[NB + 1 + NB:NB + 1 + 2 * NB])
        semi = list(rest[NB + 1 + 2 * NB:])
        c = lax.axis_index("c")
        s = lax.axis_index("s")

        def run(h_hbm, o_hbm):
            base = s * R0
            pltpu.sync_copy(h_hbm.at[pl.ds(base, R0)],
                            agg_sh.at[pl.ds(base, R0)])
            if rem:
                @pl.when(s == NS - 1)
                def _():
                    pltpu.sync_copy(h_hbm.at[pl.ds(R0 * NS, rem)],
                                    agg_sh.at[pl.ds(R0 * NS, rem)])
            plsc.subcore_barrier()

            src_s = src_hbm.at[s]
            dst_s = dst_hbm.at[s]
            srcs = [src_v.at[t] for t in range(NI)]
            dsts = [dst_v.at[t] for t in range(NI)]

            def issue_idx(j, i):
                pltpu.async_copy(src_s.at[j], srcs[i], semi[i])
                pltpu.async_copy(dst_s.at[j], dsts[i], semi[i])

            def wait_idx(j, i):
                pltpu.make_async_copy(src_s.at[j], srcs[i], semi[i]).wait()
                pltpu.make_async_copy(dst_s.at[j], dsts[i], semi[i]).wait()

            def wait_scatter(b, i):
                pltpu.make_async_copy(rows[b], agg_sh.at[dsts[i]],
                                      semsc[b]).wait()

            for t in range(min(6, CH)):
                issue_idx(t, t)
            for t in range(min(2, CH)):
                wait_idx(t, t)
                pltpu.async_copy(h_hbm.at[srcs[t]], rows[t], semg[t])

            def group(g, carry):
                for u in range(NI):
                    j = g * NI + u
                    b = u % NB
                    b2 = (u + 2) % NB
                    i2 = (u + 2) % NI
                    i6 = (u + 6) % NI

                    @pl.when(j < CH)
                    def _(j=j, u=u, b=b, b2=b2, i2=i2, i6=i6):
                        @pl.when(j >= 2)
                        def _():
                            wait_scatter(b2, (u - 2) % NI)

                        @pl.when(j + 6 < CH)
                        def _():
                            issue_idx(j + 6, i6)

                        @pl.when(j + 2 < CH)
                        def _():
                            wait_idx(j + 2, i2)
                            pltpu.async_copy(h_hbm.at[srcs[i2]],
                                             rows[b2], semg[b2])

                        pltpu.make_async_copy(h_hbm.at[srcs[u]],
                                              rows[b], semg[b]).wait()
                        pltpu.async_copy(rows[b], agg_sh.at[dsts[u]],
                                         semsc[b], add=True)

                return carry

            lax.fori_loop(0, (CH + NI - 1) // NI, group, 0)
            if CH >= 2:
                wait_scatter((CH - 2) % NB, (CH - 2) % NI)
            wait_scatter((CH - 1) % NB, (CH - 1) % NI)
            plsc.subcore_barrier()
            pltpu.sync_copy(agg_sh.at[pl.ds(base, R0)],
                            o_hbm.at[pl.ds(base, R0)])
            if rem:
                @pl.when(s == NS - 1)
                def _():
                    pltpu.sync_copy(agg_sh.at[pl.ds(R0 * NS, rem)],
                                    o_hbm.at[pl.ds(R0 * NS, rem)])

        @pl.when(c == 0)
        def _():
            run(h0, o0)

        @pl.when(c == 1)
        def _():
            run(h1, o1)

    return sc_agg


def kernel(features, edge_list, W_enc, b_enc, W_comb, b_comb, W_out, b_out):
    N = features.shape[0]
    Dlat = W_enc.shape[1]
    Lh = Dlat // 2
    E = edge_list.shape[0]
    layers = W_comb.shape[0]
    CH = -(-E // (NS * K))
    Epad = NS * K * CH

    dst = edge_list[:, 0].astype(jnp.int32)
    src = edge_list[:, 1].astype(jnp.int32)
    pad = Epad - E
    src_p = jnp.concatenate(
        [src, jnp.zeros((pad,), jnp.int32)]).reshape(NS, CH, K)
    dst_p = jnp.concatenate(
        [dst, jnp.full((pad,), N, jnp.int32)]).reshape(NS, CH, K)

    sc_agg = _make_sc_agg(N, Lh, CH)

    h0, h1 = _mm((features,), W_enc, b_enc, relu=False, split=True)
    for l in range(layers):
        s0, s1 = sc_agg(h0, h1, src_p, dst_p)
        h0, h1 = _mm((s0, s1), W_comb[l], b_comb[l], relu=True, split=True)
    return _mm((h0, h1), W_out, b_out, relu=False, split=False)
